# baseline (speedup 1.0000x reference)
import functools

import jax
import jax.numpy as jnp
from jax.experimental import pallas as pl
from jax.experimental.pallas import tpu as pltpu

_VMEM_LIMIT = 48 * 1024 * 1024


def _cparams():
    return pltpu.CompilerParams(
        dimension_semantics=("parallel", "parallel"),
        vmem_limit_bytes=_VMEM_LIMIT,
        internal_scratch_in_bytes=8 * 1024 * 1024)


def _conv_kn(fold, w_ref, bias_ref, th, w, cout):
    z = jnp.dot(fold.reshape(th * (w + 2), fold.shape[-1]), w_ref[...],
                preferred_element_type=jnp.float32)
    z = z.reshape(th, w + 2, 3 * cout)
    o = (z[:, 0:w, 0:cout] + z[:, 1:w + 1, cout:2 * cout]
         + z[:, 2:w + 2, 2 * cout:] + bias_ref[...])
    return jnp.maximum(o, 0.0).astype(jnp.bfloat16)


def _conv_k(fold, w_ref, bias_ref, th, w, cout):
    k3 = fold.shape[-1]
    acc = jnp.dot(fold[:, 0:w].reshape(th * w, k3), w_ref[0],
                  preferred_element_type=jnp.float32)
    acc = acc + jnp.dot(fold[:, 1:w + 1].reshape(th * w, k3), w_ref[1],
                        preferred_element_type=jnp.float32)
    acc = acc + jnp.dot(fold[:, 2:w + 2].reshape(th * w, k3), w_ref[2],
                        preferred_element_type=jnp.float32)
    o = jnp.maximum(acc + bias_ref[...], 0.0).astype(jnp.bfloat16)
    return o.reshape(th, w, cout)


def _pool_h(o, th):
    op = o.reshape(th // 2, 2, o.shape[1], o.shape[2])
    return jnp.maximum(op[:, 0], op[:, 1])


def _sse_partial_2d(oP, oG):
    d = oP.astype(jnp.float32) - oG.astype(jnp.float32)
    dd = d * d
    m, c = dd.shape
    return jnp.sum(dd.reshape(m // 8, 8, c), axis=0)


def _sse_partial_3d(oP, oG):
    d = oP.astype(jnp.float32) - oG.astype(jnp.float32)
    s1 = jnp.sum(d * d, axis=0)
    w_, c = s1.shape
    return jnp.sum(s1.reshape(w_ // 8, 8, c), axis=0)


def _pair_kernel(aP, hP, aG, hG, w_ref, b_ref, *outs,
                 variant, th, w, cout, pool, sse, keep, wpair):
    def band(a_ref, h_ref):
        b = jnp.concatenate([a_ref[0], h_ref[0]], axis=0)
        if wpair:
            c = b.shape[-1] // 2
            b = jnp.maximum(b[..., :c], b[..., c:])
        return jnp.concatenate([b[0:th], b[1:th + 1], b[2:th + 2]], axis=-1)

    foldP = band(aP, hP)
    foldG = band(aG, hG)
    conv = _conv_kn if variant == "B" else _conv_k
    oP = conv(foldP, w_ref, b_ref, th, w, cout)
    oG = conv(foldG, w_ref, b_ref, th, w, cout)
    i = 0
    if keep:
        if pool:
            outs[i][0] = _pool_h(oP, th)
            outs[i + 1][0] = _pool_h(oG, th)
        else:
            outs[i][0] = oP
            outs[i + 1][0] = oG
        i += 2
    if sse:
        if variant == "B":
            outs[i][0] = _sse_partial_3d(oP, oG)
        else:
            outs[i][0] = _sse_partial_2d(
                oP.reshape(th * w, cout), oG.reshape(th * w, cout))


def _conv_pair(xP, xG, w_oihw, bias, *, variant, th,
               pool=False, sse=False, keep=True, wpair=False):
    n, h, w, clanes = xP.shape
    cin = clanes // 2 if wpair else clanes
    cout = w_oihw.shape[0]
    th = min(th, h)
    nb = h // th
    pad = ((0, 0), (1, th - 1), (1, 1), (0, 0))
    xPp = jnp.pad(xP, pad)
    xGp = jnp.pad(xG, pad)

    if variant == "B":
        wf = jnp.transpose(w_oihw, (2, 1, 3, 0)).reshape(
            3 * cin, 3 * cout).astype(jnp.bfloat16)
        wspec = pl.BlockSpec((3 * cin, 3 * cout), lambda b, i: (0, 0))
    else:
        wf = jnp.transpose(w_oihw, (3, 2, 1, 0)).reshape(
            3, 3 * cin, cout).astype(jnp.bfloat16)
        wspec = pl.BlockSpec((3, 3 * cin, cout), lambda b, i: (0, 0, 0))
    b2 = bias.reshape(1, cout).astype(jnp.float32)

    band = pl.BlockSpec((1, th, w + 2, clanes), lambda b, i: (b, i, 0, 0))
    halo = pl.BlockSpec((1, 2, w + 2, clanes),
                        lambda b, i: (b, (i + 1) * (th // 2), 0, 0))

    out_shapes, out_specs = [], []
    if keep:
        if pool:
            oshape = (n, h // 2, w, cout)
            ospec = pl.BlockSpec((1, th // 2, w, cout),
                                 lambda b, i: (b, i, 0, 0))
        else:
            oshape = (n, h, w, cout)
            ospec = pl.BlockSpec((1, th, w, cout), lambda b, i: (b, i, 0, 0))
        out_shapes += [jax.ShapeDtypeStruct(oshape, jnp.bfloat16)] * 2
        out_specs += [ospec, ospec]
    if sse:
        out_shapes.append(jax.ShapeDtypeStruct((n * nb, 8, cout), jnp.float32))
        out_specs.append(
            pl.BlockSpec((1, 8, cout), lambda b, i: (b * nb + i, 0, 0)))

    body = functools.partial(_pair_kernel, variant=variant, th=th, w=w,
                             cout=cout, pool=pool, sse=sse, keep=keep,
                             wpair=wpair)
    return pl.pallas_call(
        body,
        out_shape=tuple(out_shapes),
        grid_spec=pltpu.PrefetchScalarGridSpec(
            num_scalar_prefetch=0,
            grid=(n, nb),
            in_specs=[band, halo, band, halo, wspec,
                      pl.BlockSpec((1, cout), lambda b, i: (0, 0))],
            out_specs=tuple(out_specs),
        ),
        compiler_params=_cparams(),
    )(xPp, xPp, xGp, xGp, wf, b2)


def kernel(pred_im, gt,
           conv1_1_w, conv1_1_b, conv1_2_w, conv1_2_b,
           conv2_1_w, conv2_1_b, conv2_2_w, conv2_2_b,
           conv3_1_w, conv3_1_b, conv3_2_w, conv3_2_b,
           conv3_3_w, conv3_3_b):
    xP = jnp.transpose(pred_im, (0, 2, 3, 1)).astype(jnp.bfloat16)
    xG = jnp.transpose(gt, (0, 2, 3, 1)).astype(jnp.bfloat16)
    n = xP.shape[0]

    def pair_view(x):
        nn, hh, ww, cc = x.shape
        return x.reshape(nn, hh, ww // 2, 2 * cc)

    yP, yG = _conv_pair(xP, xG, conv1_1_w, conv1_1_b, variant="B", th=8)
    pP, pG, s1 = _conv_pair(yP, yG, conv1_2_w, conv1_2_b, variant="B", th=8,
                            pool=True, sse=True)
    yP, yG = _conv_pair(pair_view(pP), pair_view(pG), conv2_1_w, conv2_1_b,
                        variant="B", th=16, wpair=True)
    pP, pG, s2 = _conv_pair(yP, yG, conv2_2_w, conv2_2_b, variant="B", th=16,
                            pool=True, sse=True)
    yP, yG = _conv_pair(pair_view(pP), pair_view(pG), conv3_1_w, conv3_1_b,
                        variant="A", th=16, wpair=True)
    yP, yG = _conv_pair(yP, yG, conv3_2_w, conv3_2_b, variant="A", th=16)
    (s3,) = _conv_pair(yP, yG, conv3_3_w, conv3_3_b, variant="A", th=16,
                       sse=True, keep=False)

    h, w = xP.shape[1], xP.shape[2]
    n1 = n * h * w * conv1_2_w.shape[0]
    n2 = n * (h // 2) * (w // 2) * conv2_2_w.shape[0]
    n3 = n * (h // 4) * (w // 4) * conv3_3_w.shape[0]
    return (jnp.sum(s1) / n1 + jnp.sum(s2) / n2 + jnp.sum(s3) / n3) / 3.0


# speedup vs baseline: 2.1455x; 2.1455x over previous
import functools

import jax
import jax.numpy as jnp
from jax.experimental import pallas as pl
from jax.experimental.pallas import tpu as pltpu

_VMEM_LIMIT = 48 * 1024 * 1024


def _cparams():
    return pltpu.CompilerParams(
        dimension_semantics=("parallel", "parallel"),
        vmem_limit_bytes=_VMEM_LIMIT,
        internal_scratch_in_bytes=8 * 1024 * 1024)


def _layer_kernel(tP, aP, bP, tG, aG, bG, w_ref, b_ref, *outs,
                  th, wp, wdata, cout, pool, sse, keep, wpair, nb):
    i = pl.program_id(1)
    m2 = th * wp

    def build(t_ref, a_ref, bo_ref):
        top = jnp.where(i > 0, t_ref[0], jnp.zeros_like(t_ref[0]))
        bot = jnp.where(i < nb - 1, bo_ref[0], jnp.zeros_like(bo_ref[0]))
        band = jnp.concatenate([top, a_ref[0], bot], axis=0)
        if wpair:
            c = band.shape[-1] // 2
            band = jnp.maximum(band[:, :c], band[:, c:])
        return jnp.concatenate(
            [band[0:m2], band[wp:wp + m2], band[2 * wp:2 * wp + m2]], axis=1)

    def conv(fold):
        z = jnp.dot(fold, w_ref[...], preferred_element_type=jnp.float32)
        zb0, zb1, zb2 = z[:, :cout], z[:, cout:2 * cout], z[:, 2 * cout:]
        o = (pltpu.roll(zb0, 1, axis=0) + zb1
             + pltpu.roll(zb2, m2 - 1, axis=0) + b_ref[...])
        o = jnp.maximum(o, 0.0).astype(jnp.bfloat16)
        o3 = o.reshape(th, wp, cout)
        mask = jax.lax.broadcasted_iota(jnp.int32, (th, wp, cout), 1) < wdata
        return jnp.where(mask, o3, jnp.zeros_like(o3))

    oP = conv(build(tP, aP, bP))
    oG = conv(build(tG, aG, bG))
    j = 0
    if keep:
        if pool:
            pP = oP.reshape(th // 2, 2, wp, cout)
            pG = oG.reshape(th // 2, 2, wp, cout)
            outs[j][0] = jnp.maximum(pP[:, 0], pP[:, 1]).reshape(
                (th // 2) * wp, cout)
            outs[j + 1][0] = jnp.maximum(pG[:, 0], pG[:, 1]).reshape(
                (th // 2) * wp, cout)
        else:
            outs[j][0] = oP.reshape(m2, cout)
            outs[j + 1][0] = oG.reshape(m2, cout)
        j += 2
    if sse:
        d = oP.astype(jnp.float32) - oG.astype(jnp.float32)
        s1 = jnp.sum(d * d, axis=0)
        outs[j][0] = jnp.sum(s1.reshape(wp // 8, 8, cout), axis=0)


def _conv_pair(xP, xG, w_oihw, bias, *, h, wp, wdata, th,
               pool=False, sse=False, keep=True, wpair=False):
    n = xP.shape[0]
    clanes = xP.shape[2]
    cin = clanes // 2 if wpair else clanes
    cout = w_oihw.shape[0]
    th = min(th, h)
    nb = h // th

    wf = jnp.transpose(w_oihw, (2, 1, 3, 0)).reshape(
        3 * cin, 3 * cout).astype(jnp.bfloat16)
    b2 = bias.reshape(1, cout).astype(jnp.float32)

    top = pl.BlockSpec((1, wp, clanes),
                       lambda b, i: (b, jnp.maximum(i * th - 1, 0), 0))
    main = pl.BlockSpec((1, th * wp, clanes), lambda b, i: (b, i, 0))
    bot = pl.BlockSpec((1, wp, clanes),
                       lambda b, i: (b, jnp.minimum((i + 1) * th, h - 1), 0))

    out_shapes, out_specs = [], []
    if keep:
        rows = (th // 2) * wp if pool else th * wp
        hrows = (h // 2) * wp if pool else h * wp
        out_shapes += [jax.ShapeDtypeStruct((n, hrows, cout), jnp.bfloat16)] * 2
        ospec = pl.BlockSpec((1, rows, cout), lambda b, i: (b, i, 0))
        out_specs += [ospec, ospec]
    if sse:
        out_shapes.append(jax.ShapeDtypeStruct((n * nb, 8, cout), jnp.float32))
        out_specs.append(
            pl.BlockSpec((1, 8, cout), lambda b, i: (b * nb + i, 0, 0)))

    body = functools.partial(_layer_kernel, th=th, wp=wp, wdata=wdata,
                             cout=cout, pool=pool, sse=sse, keep=keep,
                             wpair=wpair, nb=nb)
    return pl.pallas_call(
        body,
        out_shape=tuple(out_shapes),
        grid_spec=pltpu.PrefetchScalarGridSpec(
            num_scalar_prefetch=0,
            grid=(n, nb),
            in_specs=[top, main, bot, top, main, bot,
                      pl.BlockSpec((3 * cin, 3 * cout), lambda b, i: (0, 0)),
                      pl.BlockSpec((1, cout), lambda b, i: (0, 0))],
            out_specs=tuple(out_specs),
        ),
        compiler_params=_cparams(),
    )(xP, xP, xP, xG, xG, xG, wf, b2)


def _pair_view(x, h, wp):
    n, _, c = x.shape
    return x.reshape(n, h, wp // 2, 2 * c).reshape(n, h * (wp // 2), 2 * c)


def kernel(pred_im, gt,
           conv1_1_w, conv1_1_b, conv1_2_w, conv1_2_b,
           conv2_1_w, conv2_1_b, conv2_2_w, conv2_2_b,
           conv3_1_w, conv3_1_b, conv3_2_w, conv3_2_b,
           conv3_3_w, conv3_3_b):
    n, _, h, w = pred_im.shape
    wp1 = ((w + 2) + 15) // 16 * 16 + 16
    xP = jnp.transpose(pred_im, (0, 2, 3, 1)).astype(jnp.bfloat16)
    xG = jnp.transpose(gt, (0, 2, 3, 1)).astype(jnp.bfloat16)
    xP = jnp.pad(xP, ((0, 0), (0, 0), (0, wp1 - w), (0, 0)))
    xG = jnp.pad(xG, ((0, 0), (0, 0), (0, wp1 - w), (0, 0)))
    xP = xP.reshape(n, h * wp1, 3)
    xG = xG.reshape(n, h * wp1, 3)
    wp2, wp3 = wp1 // 2, wp1 // 4

    yP, yG = _conv_pair(xP, xG, conv1_1_w, conv1_1_b,
                        h=h, wp=wp1, wdata=w, th=8)
    pP, pG, s1 = _conv_pair(yP, yG, conv1_2_w, conv1_2_b,
                            h=h, wp=wp1, wdata=w, th=8, pool=True, sse=True)
    yP, yG = _conv_pair(_pair_view(pP, h // 2, wp1), _pair_view(pG, h // 2, wp1),
                        conv2_1_w, conv2_1_b,
                        h=h // 2, wp=wp2, wdata=w // 2, th=16, wpair=True)
    pP, pG, s2 = _conv_pair(yP, yG, conv2_2_w, conv2_2_b,
                            h=h // 2, wp=wp2, wdata=w // 2, th=16,
                            pool=True, sse=True)
    yP, yG = _conv_pair(_pair_view(pP, h // 4, wp2), _pair_view(pG, h // 4, wp2),
                        conv3_1_w, conv3_1_b,
                        h=h // 4, wp=wp3, wdata=w // 4, th=16, wpair=True)
    yP, yG = _conv_pair(yP, yG, conv3_2_w, conv3_2_b,
                        h=h // 4, wp=wp3, wdata=w // 4, th=16)
    (s3,) = _conv_pair(yP, yG, conv3_3_w, conv3_3_b,
                       h=h // 4, wp=wp3, wdata=w // 4, th=16,
                       sse=True, keep=False)

    n1 = n * h * w * conv1_2_w.shape[0]
    n2 = n * (h // 2) * (w // 2) * conv2_2_w.shape[0]
    n3 = n * (h // 4) * (w // 4) * conv3_3_w.shape[0]
    return (jnp.sum(s1) / n1 + jnp.sum(s2) / n2 + jnp.sum(s3) / n3) / 3.0


# speedup vs baseline: 2.2398x; 1.0440x over previous
import functools

import jax
import jax.numpy as jnp
from jax.experimental import pallas as pl
from jax.experimental.pallas import tpu as pltpu

_VMEM_LIMIT = 48 * 1024 * 1024


def _cparams():
    return pltpu.CompilerParams(
        dimension_semantics=("parallel", "parallel"),
        vmem_limit_bytes=_VMEM_LIMIT,
        internal_scratch_in_bytes=8 * 1024 * 1024)


def _layer_kernel(tP, aP, bP, tG, aG, bG, w_ref, b_ref, *outs,
                  th, wp, wdata, cout, pool, sse, keep, wpair, nb):
    i = pl.program_id(1)
    m2 = th * wp

    def build(t_ref, a_ref, bo_ref):
        top = jnp.where(i > 0, t_ref[0], jnp.zeros_like(t_ref[0]))
        bot = jnp.where(i < nb - 1, bo_ref[0], jnp.zeros_like(bo_ref[0]))
        band = jnp.concatenate([top, a_ref[0], bot], axis=0)
        if wpair:
            c = band.shape[-1] // 2
            band = jnp.maximum(band[:, :c], band[:, c:])
        return jnp.concatenate(
            [band[0:m2], band[wp:wp + m2], band[2 * wp:2 * wp + m2]], axis=1)

    def conv(fold):
        z = jnp.dot(fold, w_ref[...], preferred_element_type=jnp.float32)
        zb0, zb1, zb2 = z[:, :cout], z[:, cout:2 * cout], z[:, 2 * cout:]
        o = (pltpu.roll(zb0, 1, axis=0) + zb1
             + pltpu.roll(zb2, m2 - 1, axis=0) + b_ref[...])
        o = jnp.maximum(o, 0.0).astype(jnp.bfloat16)
        o3 = o.reshape(th, wp, cout)
        mask = jax.lax.broadcasted_iota(jnp.int32, (th, wp, cout), 1) < wdata
        return jnp.where(mask, o3, jnp.zeros_like(o3))

    oP = conv(build(tP, aP, bP))
    oG = conv(build(tG, aG, bG))
    j = 0
    if keep:
        if pool:
            pP = oP.reshape(th // 2, 2, wp, cout)
            pG = oG.reshape(th // 2, 2, wp, cout)
            outs[j][0] = jnp.maximum(pP[:, 0], pP[:, 1]).reshape(
                (th // 2) * wp, cout)
            outs[j + 1][0] = jnp.maximum(pG[:, 0], pG[:, 1]).reshape(
                (th // 2) * wp, cout)
        else:
            outs[j][0] = oP.reshape(m2, cout)
            outs[j + 1][0] = oG.reshape(m2, cout)
        j += 2
    if sse:
        d = oP.astype(jnp.float32) - oG.astype(jnp.float32)
        s1 = jnp.sum(d * d, axis=0)
        outs[j][0] = jnp.sum(s1.reshape(wp // 8, 8, cout), axis=0)


def _conv_pair(xP, xG, w_oihw, bias, *, h, wp, wdata, th,
               pool=False, sse=False, keep=True, wpair=False):
    n = xP.shape[0]
    clanes = xP.shape[2]
    cin = clanes // 2 if wpair else clanes
    cout = w_oihw.shape[0]
    th = min(th, h)
    nb = h // th

    wf = jnp.transpose(w_oihw, (2, 1, 3, 0)).reshape(
        3 * cin, 3 * cout).astype(jnp.bfloat16)
    b2 = bias.reshape(1, cout).astype(jnp.float32)

    top = pl.BlockSpec((1, wp, clanes),
                       lambda b, i: (b, jnp.maximum(i * th - 1, 0), 0))
    main = pl.BlockSpec((1, th * wp, clanes), lambda b, i: (b, i, 0))
    bot = pl.BlockSpec((1, wp, clanes),
                       lambda b, i: (b, jnp.minimum((i + 1) * th, h - 1), 0))

    out_shapes, out_specs = [], []
    if keep:
        rows = (th // 2) * wp if pool else th * wp
        hrows = (h // 2) * wp if pool else h * wp
        out_shapes += [jax.ShapeDtypeStruct((n, hrows, cout), jnp.bfloat16)] * 2
        ospec = pl.BlockSpec((1, rows, cout), lambda b, i: (b, i, 0))
        out_specs += [ospec, ospec]
    if sse:
        out_shapes.append(jax.ShapeDtypeStruct((n * nb, 8, cout), jnp.float32))
        out_specs.append(
            pl.BlockSpec((1, 8, cout), lambda b, i: (b * nb + i, 0, 0)))

    body = functools.partial(_layer_kernel, th=th, wp=wp, wdata=wdata,
                             cout=cout, pool=pool, sse=sse, keep=keep,
                             wpair=wpair, nb=nb)
    return pl.pallas_call(
        body,
        out_shape=tuple(out_shapes),
        grid_spec=pltpu.PrefetchScalarGridSpec(
            num_scalar_prefetch=0,
            grid=(n, nb),
            in_specs=[top, main, bot, top, main, bot,
                      pl.BlockSpec((3 * cin, 3 * cout), lambda b, i: (0, 0)),
                      pl.BlockSpec((1, cout), lambda b, i: (0, 0))],
            out_specs=tuple(out_specs),
        ),
        compiler_params=_cparams(),
    )(xP, xP, xP, xG, xG, xG, wf, b2)


def _pair_view(x, h, wp):
    n, _, c = x.shape
    return x.reshape(n, h, wp // 2, 2 * c).reshape(n, h * (wp // 2), 2 * c)


def kernel(pred_im, gt,
           conv1_1_w, conv1_1_b, conv1_2_w, conv1_2_b,
           conv2_1_w, conv2_1_b, conv2_2_w, conv2_2_b,
           conv3_1_w, conv3_1_b, conv3_2_w, conv3_2_b,
           conv3_3_w, conv3_3_b):
    n, _, h, w = pred_im.shape
    wp1 = ((w + 2) + 15) // 16 * 16 + 16
    xP = jnp.transpose(pred_im, (0, 2, 3, 1)).astype(jnp.bfloat16)
    xG = jnp.transpose(gt, (0, 2, 3, 1)).astype(jnp.bfloat16)
    xP = jnp.pad(xP, ((0, 0), (0, 0), (0, wp1 - w), (0, 0)))
    xG = jnp.pad(xG, ((0, 0), (0, 0), (0, wp1 - w), (0, 0)))
    xP = xP.reshape(n, h * wp1, 3)
    xG = xG.reshape(n, h * wp1, 3)
    wp2, wp3 = wp1 // 2, wp1 // 4

    yP, yG = _conv_pair(xP, xG, conv1_1_w, conv1_1_b,
                        h=h, wp=wp1, wdata=w, th=16)
    pP, pG, s1 = _conv_pair(yP, yG, conv1_2_w, conv1_2_b,
                            h=h, wp=wp1, wdata=w, th=16, pool=True, sse=True)
    yP, yG = _conv_pair(_pair_view(pP, h // 2, wp1), _pair_view(pG, h // 2, wp1),
                        conv2_1_w, conv2_1_b,
                        h=h // 2, wp=wp2, wdata=w // 2, th=32, wpair=True)
    pP, pG, s2 = _conv_pair(yP, yG, conv2_2_w, conv2_2_b,
                            h=h // 2, wp=wp2, wdata=w // 2, th=32,
                            pool=True, sse=True)
    yP, yG = _conv_pair(_pair_view(pP, h // 4, wp2), _pair_view(pG, h // 4, wp2),
                        conv3_1_w, conv3_1_b,
                        h=h // 4, wp=wp3, wdata=w // 4, th=32, wpair=True)
    yP, yG = _conv_pair(yP, yG, conv3_2_w, conv3_2_b,
                        h=h // 4, wp=wp3, wdata=w // 4, th=32)
    (s3,) = _conv_pair(yP, yG, conv3_3_w, conv3_3_b,
                       h=h // 4, wp=wp3, wdata=w // 4, th=32,
                       sse=True, keep=False)

    n1 = n * h * w * conv1_2_w.shape[0]
    n2 = n * (h // 2) * (w // 2) * conv2_2_w.shape[0]
    n3 = n * (h // 4) * (w // 4) * conv3_3_w.shape[0]
    return (jnp.sum(s1) / n1 + jnp.sum(s2) / n2 + jnp.sum(s3) / n3) / 3.0


# speedup vs baseline: 2.9799x; 1.3304x over previous
import functools

import jax
import jax.numpy as jnp
from jax.experimental import pallas as pl
from jax.experimental.pallas import tpu as pltpu

_VMEM_LIMIT = 48 * 1024 * 1024


def _cparams():
    return pltpu.CompilerParams(
        dimension_semantics=("parallel", "parallel"),
        vmem_limit_bytes=_VMEM_LIMIT,
        internal_scratch_in_bytes=8 * 1024 * 1024)


def _layer_kernel(tP, aP, bP, tG, aG, bG, w_ref, b_ref, *outs,
                  th, wp, wdata, cout, pool, sse, keep, wpair, nb, nchw):
    i = pl.program_id(1)
    m2 = th * wp

    def build(t_ref, a_ref, bo_ref):
        if nchw:
            top = jnp.where(i > 0, t_ref[0], jnp.zeros_like(t_ref[0]))
            bot = jnp.where(i < nb - 1, bo_ref[0], jnp.zeros_like(bo_ref[0]))
            x3 = jnp.concatenate([top, a_ref[0], bot], axis=1)
            xt = jnp.transpose(x3).astype(jnp.bfloat16)
            x4 = xt.reshape(th + 2, wdata, 3)
            band = jnp.concatenate(
                [x4, jnp.zeros((th + 2, wp - wdata, 3), jnp.bfloat16)],
                axis=1).reshape((th + 2) * wp, 3)
        else:
            top = jnp.where(i > 0, t_ref[0], jnp.zeros_like(t_ref[0]))
            bot = jnp.where(i < nb - 1, bo_ref[0], jnp.zeros_like(bo_ref[0]))
            band = jnp.concatenate([top, a_ref[0], bot], axis=0)
            if wpair:
                c = band.shape[-1] // 2
                band = jnp.maximum(band[:, :c], band[:, c:])
        return jnp.concatenate(
            [band[0:m2], band[wp:wp + m2], band[2 * wp:2 * wp + m2]], axis=1)

    def conv(fold):
        z = jnp.dot(fold, w_ref[...], preferred_element_type=jnp.float32)
        zb0, zb1, zb2 = z[:, :cout], z[:, cout:2 * cout], z[:, 2 * cout:]
        o = (pltpu.roll(zb0, 1, axis=0) + zb1
             + pltpu.roll(zb2, m2 - 1, axis=0) + b_ref[...])
        o = jnp.maximum(o, 0.0).astype(jnp.bfloat16)
        o3 = o.reshape(th, wp, cout)
        mask = jax.lax.broadcasted_iota(jnp.int32, (th, wp, cout), 1) < wdata
        return jnp.where(mask, o3, jnp.zeros_like(o3))

    oP = conv(build(tP, aP, bP))
    oG = conv(build(tG, aG, bG))
    j = 0
    if keep:
        if pool:
            pP = oP.reshape(th // 2, 2, wp, cout)
            pG = oG.reshape(th // 2, 2, wp, cout)
            outs[j][0] = jnp.maximum(pP[:, 0], pP[:, 1]).reshape(
                (th // 2) * wp, cout)
            outs[j + 1][0] = jnp.maximum(pG[:, 0], pG[:, 1]).reshape(
                (th // 2) * wp, cout)
        else:
            outs[j][0] = oP.reshape(m2, cout)
            outs[j + 1][0] = oG.reshape(m2, cout)
        j += 2
    if sse:
        d = oP.astype(jnp.float32) - oG.astype(jnp.float32)
        s1 = jnp.sum(d * d, axis=0)
        outs[j][0] = jnp.sum(s1.reshape(wp // 8, 8, cout), axis=0)


def _conv_pair(xP, xG, w_oihw, bias, *, h, wp, wdata, th,
               pool=False, sse=False, keep=True, wpair=False, nchw=False):
    n = xP.shape[0]
    cout = w_oihw.shape[0]
    th = min(th, h)
    nb = h // th

    if nchw:
        cin = xP.shape[1]
        top = pl.BlockSpec((1, cin, wdata),
                           lambda b, i: (b, 0, jnp.maximum(i * th - 1, 0)))
        main = pl.BlockSpec((1, cin, th * wdata), lambda b, i: (b, 0, i))
        bot = pl.BlockSpec((1, cin, wdata),
                           lambda b, i: (b, 0, jnp.minimum((i + 1) * th, h - 1)))
    else:
        clanes = xP.shape[2]
        cin = clanes // 2 if wpair else clanes
        top = pl.BlockSpec((1, wp, clanes),
                           lambda b, i: (b, jnp.maximum(i * th - 1, 0), 0))
        main = pl.BlockSpec((1, th * wp, clanes), lambda b, i: (b, i, 0))
        bot = pl.BlockSpec((1, wp, clanes),
                           lambda b, i: (b, jnp.minimum((i + 1) * th, h - 1), 0))

    wf = jnp.transpose(w_oihw, (2, 1, 3, 0)).reshape(
        3 * cin, 3 * cout).astype(jnp.bfloat16)
    b2 = bias.reshape(1, cout).astype(jnp.float32)

    out_shapes, out_specs = [], []
    if keep:
        rows = (th // 2) * wp if pool else th * wp
        hrows = (h // 2) * wp if pool else h * wp
        out_shapes += [jax.ShapeDtypeStruct((n, hrows, cout), jnp.bfloat16)] * 2
        ospec = pl.BlockSpec((1, rows, cout), lambda b, i: (b, i, 0))
        out_specs += [ospec, ospec]
    if sse:
        out_shapes.append(jax.ShapeDtypeStruct((n * nb, 8, cout), jnp.float32))
        out_specs.append(
            pl.BlockSpec((1, 8, cout), lambda b, i: (b * nb + i, 0, 0)))

    body = functools.partial(_layer_kernel, th=th, wp=wp, wdata=wdata,
                             cout=cout, pool=pool, sse=sse, keep=keep,
                             wpair=wpair, nb=nb, nchw=nchw)
    return pl.pallas_call(
        body,
        out_shape=tuple(out_shapes),
        grid_spec=pltpu.PrefetchScalarGridSpec(
            num_scalar_prefetch=0,
            grid=(n, nb),
            in_specs=[top, main, bot, top, main, bot,
                      pl.BlockSpec((3 * cin, 3 * cout), lambda b, i: (0, 0)),
                      pl.BlockSpec((1, cout), lambda b, i: (0, 0))],
            out_specs=tuple(out_specs),
        ),
        compiler_params=_cparams(),
    )(xP, xP, xP, xG, xG, xG, wf, b2)


def _pair_view(x, h, wp):
    n, _, c = x.shape
    return x.reshape(n, h, wp // 2, 2 * c).reshape(n, h * (wp // 2), 2 * c)


def kernel(pred_im, gt,
           conv1_1_w, conv1_1_b, conv1_2_w, conv1_2_b,
           conv2_1_w, conv2_1_b, conv2_2_w, conv2_2_b,
           conv3_1_w, conv3_1_b, conv3_2_w, conv3_2_b,
           conv3_3_w, conv3_3_b):
    n, nc, h, w = pred_im.shape
    wp1 = ((w + 2) + 15) // 16 * 16 + 16
    xP = pred_im.reshape(n, nc, h * w)
    xG = gt.reshape(n, nc, h * w)
    wp2, wp3 = wp1 // 2, wp1 // 4

    yP, yG = _conv_pair(xP, xG, conv1_1_w, conv1_1_b,
                        h=h, wp=wp1, wdata=w, th=16, nchw=True)
    pP, pG, s1 = _conv_pair(yP, yG, conv1_2_w, conv1_2_b,
                            h=h, wp=wp1, wdata=w, th=16, pool=True, sse=True)
    yP, yG = _conv_pair(_pair_view(pP, h // 2, wp1), _pair_view(pG, h // 2, wp1),
                        conv2_1_w, conv2_1_b,
                        h=h // 2, wp=wp2, wdata=w // 2, th=32, wpair=True)
    pP, pG, s2 = _conv_pair(yP, yG, conv2_2_w, conv2_2_b,
                            h=h // 2, wp=wp2, wdata=w // 2, th=32,
                            pool=True, sse=True)
    yP, yG = _conv_pair(_pair_view(pP, h // 4, wp2), _pair_view(pG, h // 4, wp2),
                        conv3_1_w, conv3_1_b,
                        h=h // 4, wp=wp3, wdata=w // 4, th=32, wpair=True)
    yP, yG = _conv_pair(yP, yG, conv3_2_w, conv3_2_b,
                        h=h // 4, wp=wp3, wdata=w // 4, th=32)
    (s3,) = _conv_pair(yP, yG, conv3_3_w, conv3_3_b,
                       h=h // 4, wp=wp3, wdata=w // 4, th=32,
                       sse=True, keep=False)

    n1 = n * h * w * conv1_2_w.shape[0]
    n2 = n * (h // 2) * (w // 2) * conv2_2_w.shape[0]
    n3 = n * (h // 4) * (w // 4) * conv3_3_w.shape[0]
    return (jnp.sum(s1) / n1 + jnp.sum(s2) / n2 + jnp.sum(s3) / n3) / 3.0
